# baseline (speedup 1.0000x reference)
import jax
import jax.numpy as jnp
from jax import lax
from jax.experimental import pallas as pl
from jax.experimental.pallas import tpu as pltpu
from jax.experimental.pallas import tpu_sc as plsc

M = 100000
D = 320
DP = 384
B = 16384
NW = 32
OWN = 3200
CCH = 80
GP = 8
ICH = 2048
L = 16


def _bcast_lane(v, r):
    idx = jnp.full((L,), r, jnp.int32)
    return v.at[idx].get(mode="promise_in_bounds")


def _lane_scalar(v, r):
    iota = lax.iota(jnp.int32, L)
    return jnp.sum(jnp.where(iota == r, v, 0))


def _sc_body(mem_h, cnt_h, q3_h, idx_h, out_h,
             idx_v, pk_v, chpk_v, cnt_v, cb0, cb1, gb0, gb1, gi0, gi1,
             si0, si1, so0, so1, sg0, sg1):
    cid = lax.axis_index("c")
    sid = lax.axis_index("s")
    w = sid * 2 + cid
    lo = w * OWN
    rows = jnp.minimum(OWN, M - lo)
    nch = rows // CCH
    iota = lax.iota(jnp.int32, L)

    pltpu.async_copy(mem_h.at[pl.ds(lo, CCH)], cb0, si0)

    pltpu.sync_copy(cnt_h.at[pl.ds(lo, OWN)], cnt_v)

    def scan_chunk(o, n):
        pltpu.sync_copy(idx_h.at[pl.ds(o * ICH, ICH)], idx_v)

        def scan_it(i, n2):
            c16 = idx_v[pl.ds(i * L, L)]
            owned = (c16 >= lo) & (c16 < lo + OWN)
            inc = plsc.cumsum(owned.astype(jnp.int32))
            offs = n2 + inc - owned.astype(jnp.int32)
            plsc.store_scatter(pk_v, [offs],
                               c16 * (L * 1024) + o * ICH + i * L + iota,
                               mask=owned)
            return n2 + jnp.sum(owned.astype(jnp.int32))

        return lax.fori_loop(0, ICH // L, scan_it, n)

    n_owned = lax.fori_loop(0, B // ICH, scan_chunk, 0)
    nsg = (n_owned + L - 1) // L

    def process(j, cbuf):
        clo = lo + j * CCH

        def sel(g, m):
            pk = pk_v[pl.ds(g * L, L)]
            valid = iota < (n_owned - g * L)
            cells = lax.shift_right_logical(pk, 14)
            inch = valid & (cells >= clo) & (cells < clo + CCH)
            inc = plsc.cumsum(inch.astype(jnp.int32))
            offs = m + inc - inch.astype(jnp.int32)
            plsc.store_scatter(chpk_v, [offs], pk, mask=inch)
            return m + jnp.sum(inch.astype(jnp.int32))

        m = lax.fori_loop(0, nsg, sel, 0)

        ng = (m + GP - 1) // GP

        def gissue(q, gi, gb, sem):
            chp = chpk_v[pl.ds(q * GP, L)]
            lanev = (iota < (m - q * GP)) & (iota < GP)
            pos = jnp.where(lanev, chp & (L * 1024 - 1), 0)
            plsc.store_scatter(gi, [iota], pos, mask=iota < GP)
            pltpu.async_copy(q3_h.at[gi], gb, sem)

        @pl.when(ng > 0)
        def _():
            gissue(0, gi0, gb0, sg0)

        gsets = ((gi0, gb0, sg0), (gi1, gb1, sg1))

        def gouter(qq, carry):
            for gsel in range(2):
                gi, gb, sem = gsets[gsel]
                ogi, ogb, osem = gsets[1 - gsel]
                q = qq * 2 + gsel

                @pl.when(q < ng)
                def _():
                    chp = chpk_v[pl.ds(q * GP, L)]
                    lanev = (iota < (m - q * GP)) & (iota < GP)
                    pos = jnp.where(lanev, chp & (L * 1024 - 1), 0)
                    cells = lax.shift_right_logical(
                        jnp.where(lanev, chp, clo * (L * 1024)), 14)
                    lr = cells - clo
                    rr = pos & 7
                    pltpu.make_async_copy(q3_h.at[gi], gb, sem).wait()

                    @pl.when(q + 1 < ng)
                    def _():
                        gissue(q + 1, ogi, ogb, osem)

                    lrr = lr * 8 + rr

                    dup = jnp.zeros((L,), jnp.bool_)
                    rc = jnp.where(lanev, cells, -1)
                    for s in range(1, GP):
                        sh = jnp.maximum(iota - s, 0)
                        shifted = rc.at[sh].get(mode="promise_in_bounds")
                        dup = dup | ((shifted == rc) & (iota >= s) & lanev)
                    has_dup = jnp.any(dup)

                    def apply_rows(av, iv):
                        for r in range(GP):
                            @pl.when(q * GP + r < m)
                            def _():
                                ar = _bcast_lane(av, r)
                                br = _bcast_lane(iv, r)
                                lr_s = _lane_scalar(lr, r)
                                for t in range(D // L):
                                    sl = pl.ds(t * L, L)
                                    qsl = pl.ds((t % 8) * L, L)
                                    cbuf[lr_s, sl] = cbuf[lr_s, sl] * ar + gb[r, t // 8, qsl] * br
                    @pl.when(jnp.logical_not(has_dup))
                    def _():
                        kv = plsc.load_gather(cnt_v, [jnp.where(lanev, cells - lo, 0)])
                        inv = 1.0 / (kv + 1.0)
                        a = kv * inv
                        plsc.store_scatter(cnt_v, [cells - lo], kv + 1.0,
                                           mask=lanev)
                        apply_rows(a, inv)

                    @pl.when(has_dup)
                    def _():
                        def pair(r, carry2):
                            @pl.when(q * GP + r < m)
                            def _():
                                csplat = _bcast_lane(cells, r)
                                kv = plsc.load_gather(cnt_v, [csplat - lo])
                                inv = 1.0 / (kv + 1.0)
                                a = kv * inv
                                plsc.store_scatter(cnt_v, [csplat - lo], kv + 1.0,
                                                   mask=iota == 0)
                                lr_s = _lane_scalar(lr, r)
                                for t in range(D // L):
                                    sl = pl.ds(t * L, L)
                                    qsl = pl.ds((t % 8) * L, L)
                                    cbuf[lr_s, sl] = cbuf[lr_s, sl] * a + gb[r, t // 8, qsl] * inv
                            return carry2

                        lax.fori_loop(0, GP, pair, 0)
            return carry

        lax.fori_loop(0, (ng + 1) // 2, gouter, 0)

    bufs = ((cb0, si0, so0), (cb1, si1, so1))

    def outer(jj, carry):
        for bsel in range(2):
            cbuf, si, so = bufs[bsel]
            ocbuf, osi, oso = bufs[1 - bsel]
            j = jj * 2 + bsel
            r0 = lo + j * CCH
            pltpu.make_async_copy(mem_h.at[pl.ds(r0, CCH)], cbuf, si).wait()

            @pl.when(j + 1 < nch)
            def _():
                @pl.when(j >= 1)
                def _():
                    pltpu.make_async_copy(
                        ocbuf, out_h.at[pl.ds(r0 - CCH, CCH)], oso).wait()
                pltpu.async_copy(mem_h.at[pl.ds(r0 + CCH, CCH)], ocbuf, osi)

            process(j, cbuf)
            pltpu.async_copy(cbuf, out_h.at[pl.ds(r0, CCH)], so)
        return carry

    lax.fori_loop(0, nch // 2, outer, 0)

    rlast = lo + (nch - 1) * CCH
    pltpu.make_async_copy(cb0, out_h.at[pl.ds(rlast - CCH, CCH)], so0).wait()
    pltpu.make_async_copy(cb1, out_h.at[pl.ds(rlast, CCH)], so1).wait()


@jax.jit
def kernel(mem, ent_counter, query, cell_idx):
    cnt_pad = jnp.pad(ent_counter, (0, NW * OWN - M))
    q3 = jnp.pad(query, ((0, 0), (0, 1024 - D))).reshape(B, 8, 128)
    idx32 = cell_idx.astype(jnp.int32)
    mesh = plsc.VectorSubcoreMesh(core_axis_name="c", subcore_axis_name="s",
                                  num_cores=2, num_subcores=16)
    f = pl.kernel(
        _sc_body,
        out_type=jax.ShapeDtypeStruct((M, D), jnp.float32),
        mesh=mesh,
        compiler_params=pltpu.CompilerParams(needs_layout_passes=False),
        scratch_types=[
            pltpu.VMEM((ICH,), jnp.int32),
            pltpu.VMEM((B + L,), jnp.int32),
            pltpu.VMEM((B + L,), jnp.int32),
            pltpu.VMEM((OWN,), jnp.float32),
            pltpu.VMEM((CCH, D), jnp.float32),
            pltpu.VMEM((CCH, D), jnp.float32),
            pltpu.VMEM((GP, 8, 128), jnp.float32),
            pltpu.VMEM((GP, 8, 128), jnp.float32),
            pltpu.VMEM((GP,), jnp.int32),
            pltpu.VMEM((GP,), jnp.int32),
            pltpu.SemaphoreType.DMA,
            pltpu.SemaphoreType.DMA,
            pltpu.SemaphoreType.DMA,
            pltpu.SemaphoreType.DMA,
            pltpu.SemaphoreType.DMA,
            pltpu.SemaphoreType.DMA,
        ],
    )
    return f(mem, cnt_pad, q3, idx32)


# speedup vs baseline: 1.2504x; 1.2504x over previous
import jax
import jax.numpy as jnp
from jax import lax
from jax.experimental import pallas as pl
from jax.experimental.pallas import tpu as pltpu
from jax.experimental.pallas import tpu_sc as plsc

M = 100000
D = 320
DP = 384
B = 16384
NW = 32
OWN = 3200
CCH = 80
GP = 4
ICH = 2048
L = 16


def _bcast_lane(v, r):
    idx = jnp.full((L,), r, jnp.int32)
    return v.at[idx].get(mode="promise_in_bounds")


def _lane_scalar(v, r):
    iota = lax.iota(jnp.int32, L)
    return jnp.sum(jnp.where(iota == r, v, 0))


def _sc_body(mem_h, cnt_h, q3_h, idx_h, out_h,
             idx_v, pk_v, chpk_v, cnt_v, cb0, cb1, gb0, gb1, gi0, gi1,
             si0, si1, so0, so1, sg0, sg1):
    cid = lax.axis_index("c")
    sid = lax.axis_index("s")
    w = sid * 2 + cid
    lo = w * OWN
    rows = jnp.minimum(OWN, M - lo)
    nch = rows // CCH
    iota = lax.iota(jnp.int32, L)

    pltpu.async_copy(mem_h.at[pl.ds(lo, CCH)], cb0, si0)

    pltpu.sync_copy(cnt_h.at[pl.ds(lo, OWN)], cnt_v)

    def scan_chunk(o, n):
        pltpu.sync_copy(idx_h.at[pl.ds(o * ICH, ICH)], idx_v)

        def scan_it(i, n2):
            c16 = idx_v[pl.ds(i * L, L)]
            owned = (c16 >= lo) & (c16 < lo + OWN)
            inc = plsc.cumsum(owned.astype(jnp.int32))
            offs = n2 + inc - owned.astype(jnp.int32)
            plsc.store_scatter(pk_v, [offs],
                               c16 * (L * 1024) + o * ICH + i * L + iota,
                               mask=owned)
            return n2 + jnp.sum(owned.astype(jnp.int32))

        return lax.fori_loop(0, ICH // L, scan_it, n)

    n_owned = lax.fori_loop(0, B // ICH, scan_chunk, 0)
    nsg = (n_owned + L - 1) // L

    HALF = (B + L) // 2

    def select2(jj):
        clo = lo + jj * 2 * CCH

        def sel(g, ms):
            m0, m1 = ms
            pk = pk_v[pl.ds(g * L, L)]
            valid = iota < (n_owned - g * L)
            cells = lax.shift_right_logical(pk, 14)
            in0 = valid & (cells >= clo) & (cells < clo + CCH)
            in1 = valid & (cells >= clo + CCH) & (cells < clo + 2 * CCH)
            inc0 = plsc.cumsum(in0.astype(jnp.int32))
            inc1 = plsc.cumsum(in1.astype(jnp.int32))
            plsc.store_scatter(chpk_v, [m0 + inc0 - in0.astype(jnp.int32)],
                               pk, mask=in0)
            plsc.store_scatter(chpk_v, [HALF + m1 + inc1 - in1.astype(jnp.int32)],
                               pk, mask=in1)
            return (m0 + jnp.sum(in0.astype(jnp.int32)),
                    m1 + jnp.sum(in1.astype(jnp.int32)))

        return lax.fori_loop(0, nsg, sel, (0, 0))

    def process(j, cbuf, m, base):
        clo = lo + j * CCH

        ng = (m + GP - 1) // GP

        def gissue(q, gi, gb, sem):
            chp = chpk_v[pl.ds(base + q * GP, L)]
            lanev = (iota < (m - q * GP)) & (iota < GP)
            pos = jnp.where(lanev, chp & (L * 1024 - 1), 0)
            plsc.store_scatter(gi, [iota], lax.shift_right_logical(pos, 3),
                               mask=iota < GP)
            pltpu.async_copy(q3_h.at[gi], gb, sem)

        @pl.when(ng > 0)
        def _():
            gissue(0, gi0, gb0, sg0)

        gsets = ((gi0, gb0, sg0), (gi1, gb1, sg1))

        def gouter(qq, carry):
            for gsel in range(2):
                gi, gb, sem = gsets[gsel]
                ogi, ogb, osem = gsets[1 - gsel]
                q = qq * 2 + gsel

                @pl.when(q < ng)
                def _():
                    chp = chpk_v[pl.ds(base + q * GP, L)]
                    lanev = (iota < (m - q * GP)) & (iota < GP)
                    pos = jnp.where(lanev, chp & (L * 1024 - 1), 0)
                    cells = lax.shift_right_logical(
                        jnp.where(lanev, chp, clo * (L * 1024)), 14)
                    lr = cells - clo
                    rr = pos & 7
                    pltpu.make_async_copy(q3_h.at[gi], gb, sem).wait()

                    @pl.when(q + 1 < ng)
                    def _():
                        gissue(q + 1, ogi, ogb, osem)

                    lrr = lr * 8 + rr

                    dup = jnp.zeros((L,), jnp.bool_)
                    rc = jnp.where(lanev, cells, -1)
                    for s in range(1, GP):
                        sh = jnp.maximum(iota - s, 0)
                        shifted = rc.at[sh].get(mode="promise_in_bounds")
                        dup = dup | ((shifted == rc) & (iota >= s) & lanev)
                    has_dup = jnp.any(dup)

                    def apply_rows(av, iv):
                        for r in range(GP):
                            @pl.when(q * GP + r < m)
                            def _():
                                ar = _bcast_lane(av, r)
                                br = _bcast_lane(iv, r)
                                lrr_s = _lane_scalar(lrr, r)
                                lr_s = lax.shift_right_logical(lrr_s, 3)
                                rr_s = lrr_s & 7
                                for t in range(D // L):
                                    sl = pl.ds(t * L, L)
                                    cbuf[lr_s, sl] = cbuf[lr_s, sl] * ar + gb[r, rr_s, sl] * br
                    @pl.when(jnp.logical_not(has_dup))
                    def _():
                        kv = plsc.load_gather(cnt_v, [jnp.where(lanev, cells - lo, 0)])
                        inv = 1.0 / (kv + 1.0)
                        a = kv * inv
                        plsc.store_scatter(cnt_v, [cells - lo], kv + 1.0,
                                           mask=lanev)
                        apply_rows(a, inv)

                    @pl.when(has_dup)
                    def _():
                        def pair(r, carry2):
                            @pl.when(q * GP + r < m)
                            def _():
                                csplat = _bcast_lane(cells, r)
                                kv = plsc.load_gather(cnt_v, [csplat - lo])
                                inv = 1.0 / (kv + 1.0)
                                a = kv * inv
                                plsc.store_scatter(cnt_v, [csplat - lo], kv + 1.0,
                                                   mask=iota == 0)
                                lrr_s = _lane_scalar(lrr, r)
                                lr_s = lax.shift_right_logical(lrr_s, 3)
                                rr_s = lrr_s & 7
                                for t in range(D // L):
                                    sl = pl.ds(t * L, L)
                                    cbuf[lr_s, sl] = cbuf[lr_s, sl] * a + gb[r, rr_s, sl] * inv
                            return carry2

                        lax.fori_loop(0, GP, pair, 0)
            return carry

        lax.fori_loop(0, (ng + 1) // 2, gouter, 0)

    bufs = ((cb0, si0, so0), (cb1, si1, so1))

    def outer(jj, carry):
        ms = select2(jj)
        for bsel in range(2):
            cbuf, si, so = bufs[bsel]
            ocbuf, osi, oso = bufs[1 - bsel]
            j = jj * 2 + bsel
            r0 = lo + j * CCH
            pltpu.make_async_copy(mem_h.at[pl.ds(r0, CCH)], cbuf, si).wait()

            @pl.when(j + 1 < nch)
            def _():
                @pl.when(j >= 1)
                def _():
                    pltpu.make_async_copy(
                        ocbuf, out_h.at[pl.ds(r0 - CCH, CCH)], oso).wait()
                pltpu.async_copy(mem_h.at[pl.ds(r0 + CCH, CCH)], ocbuf, osi)

            process(j, cbuf, ms[bsel], bsel * HALF)
            pltpu.async_copy(cbuf, out_h.at[pl.ds(r0, CCH)], so)
        return carry

    lax.fori_loop(0, nch // 2, outer, 0)

    rlast = lo + (nch - 1) * CCH
    pltpu.make_async_copy(cb0, out_h.at[pl.ds(rlast - CCH, CCH)], so0).wait()
    pltpu.make_async_copy(cb1, out_h.at[pl.ds(rlast, CCH)], so1).wait()


@jax.jit
def kernel(mem, ent_counter, query, cell_idx):
    cnt_pad = jnp.pad(ent_counter, (0, NW * OWN - M))
    q3 = jnp.pad(query, ((0, 0), (0, DP - D))).reshape(B // 8, 8, DP)
    idx32 = cell_idx.astype(jnp.int32)
    mesh = plsc.VectorSubcoreMesh(core_axis_name="c", subcore_axis_name="s",
                                  num_cores=2, num_subcores=16)
    f = pl.kernel(
        _sc_body,
        out_type=jax.ShapeDtypeStruct((M, D), jnp.float32),
        mesh=mesh,
        compiler_params=pltpu.CompilerParams(needs_layout_passes=False),
        scratch_types=[
            pltpu.VMEM((ICH,), jnp.int32),
            pltpu.VMEM((B + L,), jnp.int32),
            pltpu.VMEM((B + L,), jnp.int32),
            pltpu.VMEM((OWN,), jnp.float32),
            pltpu.VMEM((CCH, D), jnp.float32),
            pltpu.VMEM((CCH, D), jnp.float32),
            pltpu.VMEM((GP, 8, DP), jnp.float32),
            pltpu.VMEM((GP, 8, DP), jnp.float32),
            pltpu.VMEM((GP,), jnp.int32),
            pltpu.VMEM((GP,), jnp.int32),
            pltpu.SemaphoreType.DMA,
            pltpu.SemaphoreType.DMA,
            pltpu.SemaphoreType.DMA,
            pltpu.SemaphoreType.DMA,
            pltpu.SemaphoreType.DMA,
            pltpu.SemaphoreType.DMA,
        ],
    )
    return f(mem, cnt_pad, q3, idx32)


# speedup vs baseline: 1.3350x; 1.0676x over previous
import jax
import jax.numpy as jnp
from jax import lax
from jax.experimental import pallas as pl
from jax.experimental.pallas import tpu as pltpu
from jax.experimental.pallas import tpu_sc as plsc

M = 100000
D = 320
DP = 384
B = 16384
NW = 32
OWN = 3200
CCH = 80
GP = 2
ICH = 2048
L = 16


def _bcast_lane(v, r):
    idx = jnp.full((L,), r, jnp.int32)
    return v.at[idx].get(mode="promise_in_bounds")


def _lane_scalar(v, r):
    iota = lax.iota(jnp.int32, L)
    return jnp.sum(jnp.where(iota == r, v, 0))


def _sc_body(mem_h, cnt_h, q3_h, idx_h, out_h,
             idx_v, pk_v, chpk_v, cnt_v, cb0, cb1,
             gb0, gb1, gb2, gb3, gi0, gi1, gi2, gi3,
             si0, si1, so0, so1, sg0, sg1, sg2, sg3):
    cid = lax.axis_index("c")
    sid = lax.axis_index("s")
    w = sid * 2 + cid
    lo = w * OWN
    rows = jnp.minimum(OWN, M - lo)
    nch = rows // CCH
    iota = lax.iota(jnp.int32, L)

    pltpu.async_copy(mem_h.at[pl.ds(lo, CCH)], cb0, si0)

    pltpu.sync_copy(cnt_h.at[pl.ds(lo, OWN)], cnt_v)

    def scan_chunk(o, n):
        pltpu.sync_copy(idx_h.at[pl.ds(o * ICH, ICH)], idx_v)

        def scan_it(i, n2):
            c16 = idx_v[pl.ds(i * L, L)]
            owned = (c16 >= lo) & (c16 < lo + OWN)
            inc = plsc.cumsum(owned.astype(jnp.int32))
            offs = n2 + inc - owned.astype(jnp.int32)
            plsc.store_scatter(pk_v, [offs],
                               c16 * (L * 1024) + o * ICH + i * L + iota,
                               mask=owned)
            return n2 + jnp.sum(owned.astype(jnp.int32))

        return lax.fori_loop(0, ICH // L, scan_it, n)

    n_owned = lax.fori_loop(0, B // ICH, scan_chunk, 0)
    nsg = (n_owned + L - 1) // L

    HALF = (B + L) // 2

    def select2(jj):
        clo = lo + jj * 2 * CCH

        def sel(g, ms):
            m0, m1 = ms
            pk = pk_v[pl.ds(g * L, L)]
            valid = iota < (n_owned - g * L)
            cells = lax.shift_right_logical(pk, 14)
            in0 = valid & (cells >= clo) & (cells < clo + CCH)
            in1 = valid & (cells >= clo + CCH) & (cells < clo + 2 * CCH)
            inc0 = plsc.cumsum(in0.astype(jnp.int32))
            inc1 = plsc.cumsum(in1.astype(jnp.int32))
            plsc.store_scatter(chpk_v, [m0 + inc0 - in0.astype(jnp.int32)],
                               pk, mask=in0)
            plsc.store_scatter(chpk_v, [HALF + m1 + inc1 - in1.astype(jnp.int32)],
                               pk, mask=in1)
            return (m0 + jnp.sum(in0.astype(jnp.int32)),
                    m1 + jnp.sum(in1.astype(jnp.int32)))

        return lax.fori_loop(0, nsg, sel, (0, 0))

    def process(j, cbuf, m, base):
        clo = lo + j * CCH

        ng = (m + GP - 1) // GP

        def gissue(q, gi, gb, sem):
            chp = chpk_v[pl.ds(base + q * GP, L)]
            lanev = (iota < (m - q * GP)) & (iota < GP)
            pos = jnp.where(lanev, chp & (L * 1024 - 1), 0)
            plsc.store_scatter(gi, [iota], lax.shift_right_logical(pos, 3),
                               mask=iota < GP)
            pltpu.async_copy(q3_h.at[gi], gb, sem)

        gsets = ((gi0, gb0, sg0), (gi1, gb1, sg1),
                 (gi2, gb2, sg2), (gi3, gb3, sg3))

        for p in range(3):
            @pl.when(p < ng)
            def _(p=p):
                gissue(p, *gsets[p])

        def gouter(qq, carry):
            for gsel in range(4):
                gi, gb, sem = gsets[gsel]
                q = qq * 4 + gsel

                @pl.when(q < ng)
                def _():
                    chp = chpk_v[pl.ds(base + q * GP, L)]
                    lanev = (iota < (m - q * GP)) & (iota < GP)
                    pos = jnp.where(lanev, chp & (L * 1024 - 1), 0)
                    cells = lax.shift_right_logical(
                        jnp.where(lanev, chp, clo * (L * 1024)), 14)
                    lr = cells - clo
                    rr = pos & 7
                    pltpu.make_async_copy(q3_h.at[gi], gb, sem).wait()

                    @pl.when(q + 3 < ng)
                    def _():
                        gissue(q + 3, *gsets[(gsel + 3) % 4])

                    lrr = lr * 8 + rr

                    dup = jnp.zeros((L,), jnp.bool_)
                    rc = jnp.where(lanev, cells, -1)
                    for s in range(1, GP):
                        sh = jnp.maximum(iota - s, 0)
                        shifted = rc.at[sh].get(mode="promise_in_bounds")
                        dup = dup | ((shifted == rc) & (iota >= s) & lanev)
                    has_dup = jnp.any(dup)

                    def apply_rows(av, iv):
                        for r in range(GP):
                            @pl.when(q * GP + r < m)
                            def _():
                                ar = _bcast_lane(av, r)
                                br = _bcast_lane(iv, r)
                                lrr_s = _lane_scalar(lrr, r)
                                lr_s = lax.shift_right_logical(lrr_s, 3)
                                rr_s = lrr_s & 7
                                for t in range(D // L):
                                    sl = pl.ds(t * L, L)
                                    cbuf[lr_s, sl] = cbuf[lr_s, sl] * ar + gb[r, rr_s, sl] * br
                    @pl.when(jnp.logical_not(has_dup))
                    def _():
                        kv = plsc.load_gather(cnt_v, [jnp.where(lanev, cells - lo, 0)])
                        inv = 1.0 / (kv + 1.0)
                        a = kv * inv
                        plsc.store_scatter(cnt_v, [cells - lo], kv + 1.0,
                                           mask=lanev)
                        apply_rows(a, inv)

                    @pl.when(has_dup)
                    def _():
                        def pair(r, carry2):
                            @pl.when(q * GP + r < m)
                            def _():
                                csplat = _bcast_lane(cells, r)
                                kv = plsc.load_gather(cnt_v, [csplat - lo])
                                inv = 1.0 / (kv + 1.0)
                                a = kv * inv
                                plsc.store_scatter(cnt_v, [csplat - lo], kv + 1.0,
                                                   mask=iota == 0)
                                lrr_s = _lane_scalar(lrr, r)
                                lr_s = lax.shift_right_logical(lrr_s, 3)
                                rr_s = lrr_s & 7
                                for t in range(D // L):
                                    sl = pl.ds(t * L, L)
                                    cbuf[lr_s, sl] = cbuf[lr_s, sl] * a + gb[r, rr_s, sl] * inv
                            return carry2

                        lax.fori_loop(0, GP, pair, 0)
            return carry

        lax.fori_loop(0, (ng + 3) // 4, gouter, 0)

    bufs = ((cb0, si0, so0), (cb1, si1, so1))

    def outer(jj, carry):
        ms = select2(jj)
        for bsel in range(2):
            cbuf, si, so = bufs[bsel]
            ocbuf, osi, oso = bufs[1 - bsel]
            j = jj * 2 + bsel
            r0 = lo + j * CCH
            pltpu.make_async_copy(mem_h.at[pl.ds(r0, CCH)], cbuf, si).wait()

            @pl.when(j + 1 < nch)
            def _():
                @pl.when(j >= 1)
                def _():
                    pltpu.make_async_copy(
                        ocbuf, out_h.at[pl.ds(r0 - CCH, CCH)], oso).wait()
                pltpu.async_copy(mem_h.at[pl.ds(r0 + CCH, CCH)], ocbuf, osi)

            process(j, cbuf, ms[bsel], bsel * HALF)
            pltpu.async_copy(cbuf, out_h.at[pl.ds(r0, CCH)], so)
        return carry

    lax.fori_loop(0, nch // 2, outer, 0)

    rlast = lo + (nch - 1) * CCH
    pltpu.make_async_copy(cb0, out_h.at[pl.ds(rlast - CCH, CCH)], so0).wait()
    pltpu.make_async_copy(cb1, out_h.at[pl.ds(rlast, CCH)], so1).wait()


@jax.jit
def kernel(mem, ent_counter, query, cell_idx):
    cnt_pad = jnp.pad(ent_counter, (0, NW * OWN - M))
    q3 = jnp.pad(query, ((0, 0), (0, DP - D))).reshape(B // 8, 8, DP)
    idx32 = cell_idx.astype(jnp.int32)
    mesh = plsc.VectorSubcoreMesh(core_axis_name="c", subcore_axis_name="s",
                                  num_cores=2, num_subcores=16)
    f = pl.kernel(
        _sc_body,
        out_type=jax.ShapeDtypeStruct((M, D), jnp.float32),
        mesh=mesh,
        compiler_params=pltpu.CompilerParams(needs_layout_passes=False),
        scratch_types=[
            pltpu.VMEM((ICH,), jnp.int32),
            pltpu.VMEM((B + L,), jnp.int32),
            pltpu.VMEM((B + L,), jnp.int32),
            pltpu.VMEM((OWN,), jnp.float32),
            pltpu.VMEM((CCH, D), jnp.float32),
            pltpu.VMEM((CCH, D), jnp.float32),
            pltpu.VMEM((GP, 8, DP), jnp.float32),
            pltpu.VMEM((GP, 8, DP), jnp.float32),
            pltpu.VMEM((GP, 8, DP), jnp.float32),
            pltpu.VMEM((GP, 8, DP), jnp.float32),
            pltpu.VMEM((GP,), jnp.int32),
            pltpu.VMEM((GP,), jnp.int32),
            pltpu.VMEM((GP,), jnp.int32),
            pltpu.VMEM((GP,), jnp.int32),
            pltpu.SemaphoreType.DMA,
            pltpu.SemaphoreType.DMA,
            pltpu.SemaphoreType.DMA,
            pltpu.SemaphoreType.DMA,
            pltpu.SemaphoreType.DMA,
            pltpu.SemaphoreType.DMA,
            pltpu.SemaphoreType.DMA,
            pltpu.SemaphoreType.DMA,
        ],
    )
    return f(mem, cnt_pad, q3, idx32)
